# baseline (speedup 1.0000x reference)
import jax
import jax.numpy as jnp
from jax.experimental import pallas as pl
from jax.experimental.pallas import tpu as pltpu

N, C, L = 16, 2048, 512
CB = 512


def _tc_body(tgt_ref, s_ref, t_ref, loss_ref, inst_ref):
    n = pl.program_id(0)
    cb = pl.program_id(1)
    ncb = pl.num_programs(1)

    @pl.when(jnp.logical_and(n == 0, cb == 0))
    def _():
        loss_ref[0, 0] = 0.0

    @pl.when(cb == 0)
    def _():
        inst_ref[0, 0] = 0.0

    m = tgt_ref[n, pl.ds(cb * CB, CB)].astype(jnp.float32)[:, None]
    d = s_ref[0] - t_ref[0]
    a = jnp.abs(d)
    e = jnp.where(a < 1.0, 0.5 * d * d, a - 0.5)
    inst_ref[0, 0] += jnp.sum(e * m)

    @pl.when(cb == ncb - 1)
    def _():
        n_pos = jnp.sum(tgt_ref[n, :].astype(jnp.float32))
        w = jnp.where(n_pos > 1.0, 1.0 / (n_pos * L), 0.0)
        loss_ref[0, 0] += inst_ref[0, 0] * w


def kernel(le_student, le_teacher, targets):
    out = pl.pallas_call(
        _tc_body,
        grid=(N, C // CB),
        in_specs=[
            pl.BlockSpec((N, C), lambda n, cb: (0, 0)),
            pl.BlockSpec((1, CB, L), lambda n, cb: (n, cb, 0)),
            pl.BlockSpec((1, CB, L), lambda n, cb: (n, cb, 0)),
        ],
        out_specs=pl.BlockSpec((1, 1), lambda n, cb: (0, 0),
                               memory_space=pltpu.SMEM),
        out_shape=jax.ShapeDtypeStruct((1, 1), jnp.float32),
        scratch_shapes=[pltpu.SMEM((1, 1), jnp.float32)],
    )(targets, le_student, le_teacher)
    return out[0, 0]
